# baseline (speedup 1.0000x reference)
import functools

import jax
import jax.numpy as jnp
from jax import lax
from jax.experimental import pallas as pl
from jax.experimental.pallas import tpu as pltpu
from jax.experimental.pallas import tpu_sc as plsc

N_NODES = 100000
F_IN = 128
HID = 64
LBL = 16
ALPHA = 0.1
N_ITERS = 10

NC = 2
NS = 16
NW = NC * NS

CHUNK = 128
ROWS_PER_TILE = N_NODES // NS
STAGE = 250

EDGE_BATCH = 8
FEAT_BATCH = 8

_mesh = plsc.VectorSubcoreMesh(core_axis_name="c", subcore_axis_name="s")


def _zero_own_slice(stage_v, acc_sh, s):
    def zrow(i, _):
        stage_v[i] = jnp.zeros((LBL,), jnp.float32)
        return 0
    lax.fori_loop(0, STAGE, zrow, 0)

    def zcopy(k, _):
        pltpu.sync_copy(stage_v, acc_sh.at[pl.ds(s * ROWS_PER_TILE + k * STAGE, STAGE)])
        return 0
    lax.fori_loop(0, ROWS_PER_TILE // STAGE, zcopy, 0)


def _dump_own_slice(stage_v, acc_sh, out2d, s):
    def dcopy(k, _):
        off = s * ROWS_PER_TILE + k * STAGE
        pltpu.sync_copy(acc_sh.at[pl.ds(off, STAGE)], stage_v)
        pltpu.sync_copy(stage_v, out2d.at[pl.ds(off, STAGE)])
        return 0
    lax.fori_loop(0, ROWS_PER_TILE // STAGE, dcopy, 0)


def _pipelined_chunks(nchunks, operand, sb, gbufs, obufs, acc_sh, gsem, ssem):
    ng, no = len(gbufs), len(obufs)
    gd, sd = {}, {}
    for h in range(ng - 1):
        gd[h] = pltpu.async_copy(operand.at[sb.at[h, 0]], gbufs[h], gsem[h])
    for j in range(nchunks):
        if j >= no:
            sd[j - no].wait()
        nxt = j + ng - 1
        if nxt < nchunks:
            gd[nxt] = pltpu.async_copy(
                operand.at[sb.at[nxt, 0]], gbufs[nxt % ng], gsem[nxt % ng])
        gd[j].wait()
        gb = gbufs[j % ng]
        ob = obufs[j % no]

        def scale(g, _, gb=gb, ob=ob, j=j):
            base16 = g * 16
            wv = plsc.bitcast(sb[j, 2, pl.ds(base16, 16)], jnp.float32)
            for i in range(16):
                ob[base16 + i] = gb[base16 + i] * wv[i]
            return 0
        lax.fori_loop(0, CHUNK // 16, scale, 0)
        sd[j] = pltpu.async_copy(
            ob, acc_sh.at[sb.at[j, 1]], ssem[j % no], add=True)
    for t in range(no):
        sd[nchunks - no + t].wait()


def _batches_with_prefetch(n_batches, nchunks, stg_hbm, first_chunk_fn,
                           operand, sbufs, gbufs, obufs, acc_sh,
                           gsem, ssem, isem):
    pltpu.async_copy(
        stg_hbm.at[pl.ds(first_chunk_fn(0), nchunks)], sbufs[0], isem[0])

    def pair_body(b2, _):
        for q in range(2):
            b = b2 * 2 + q
            nb = b + 1

            @pl.when(nb < n_batches)
            def _():
                pltpu.async_copy(
                    stg_hbm.at[pl.ds(first_chunk_fn(nb), nchunks)],
                    sbufs[1 - q], isem[1 - q])
            pltpu.make_async_copy(
                stg_hbm.at[pl.ds(0, nchunks)], sbufs[q], isem[q]).wait()
            _pipelined_chunks(nchunks, operand, sbufs[q], gbufs, obufs,
                              acc_sh, gsem, ssem)
        return 0
    lax.fori_loop(0, n_batches // 2, pair_body, 0)


def _feat_body(nnz_chunks_per_tile, w1r_hbm, stg_hbm, h1_hbm,
               w1_v, sbufs, gbufs, obufs, stage_v,
               w1_sh, acc_sh, gsem, ssem, isem):
    c = lax.axis_index("c")
    s = lax.axis_index("s")
    n_batches = nnz_chunks_per_tile // FEAT_BATCH

    for p in range(2):
        fc = c * 2 + p
        _zero_own_slice(stage_v, acc_sh, s)

        @pl.when(s == 0)
        def _():
            pltpu.sync_copy(w1r_hbm.at[fc], w1_v)
            pltpu.sync_copy(w1_v, w1_sh)
        plsc.subcore_barrier()

        def first_chunk(b):
            return s * nnz_chunks_per_tile + b * FEAT_BATCH
        _batches_with_prefetch(n_batches, FEAT_BATCH, stg_hbm, first_chunk,
                               w1_sh, sbufs, gbufs, obufs, acc_sh,
                               gsem, ssem, isem)
        plsc.subcore_barrier()

        _dump_own_slice(stage_v, acc_sh, h1_hbm.at[fc], s)


def _feat_spmm(w1r, stg):
    nnz_chunks = stg.shape[0]
    per_tile = nnz_chunks // NS
    body = functools.partial(_feat_body, per_tile)
    return pl.kernel(
        body,
        out_type=jax.ShapeDtypeStruct((4, N_NODES, LBL), jnp.float32),
        mesh=_mesh,
        compiler_params=pltpu.CompilerParams(use_tc_tiling_on_sc=False, needs_layout_passes=False),
        scratch_types=[
            pltpu.VMEM((F_IN, LBL), jnp.float32),
            [pltpu.VMEM((FEAT_BATCH, 3, CHUNK), jnp.int32)] * 2,
            [pltpu.VMEM((CHUNK, LBL), jnp.float32)] * 5,
            [pltpu.VMEM((CHUNK, LBL), jnp.float32)] * 3,
            pltpu.VMEM((STAGE, LBL), jnp.float32),
            pltpu.VMEM_SHARED((F_IN, LBL), jnp.float32),
            pltpu.VMEM_SHARED((N_NODES, LBL), jnp.float32),
            [pltpu.SemaphoreType.DMA] * 5,
            [pltpu.SemaphoreType.DMA] * 3,
            [pltpu.SemaphoreType.DMA] * 2,
        ],
    )(w1r, stg)


def _edge_body(chunks_per_tile, pred_hbm, stg_hbm, part_hbm,
               sbufs, gbufs, obufs, stage_v, acc_sh, gsem, ssem, isem):
    c = lax.axis_index("c")
    s = lax.axis_index("s")
    wid = s * NC + c
    n_batches = chunks_per_tile // EDGE_BATCH

    _zero_own_slice(stage_v, acc_sh, s)
    plsc.subcore_barrier()

    def first_chunk(b):
        return wid * chunks_per_tile + b * EDGE_BATCH
    _batches_with_prefetch(n_batches, EDGE_BATCH, stg_hbm, first_chunk,
                           pred_hbm, sbufs, gbufs, obufs, acc_sh,
                           gsem, ssem, isem)
    plsc.subcore_barrier()

    _dump_own_slice(stage_v, acc_sh, part_hbm.at[c], s)


def _edge_spmm(pred, stg):
    chunks = stg.shape[0]
    per_tile = chunks // NW
    body = functools.partial(_edge_body, per_tile)
    return pl.kernel(
        body,
        out_type=jax.ShapeDtypeStruct((NC, N_NODES, LBL), jnp.float32),
        mesh=_mesh,
        compiler_params=pltpu.CompilerParams(use_tc_tiling_on_sc=False, needs_layout_passes=False),
        scratch_types=[
            [pltpu.VMEM((EDGE_BATCH, 3, CHUNK), jnp.int32)] * 2,
            [pltpu.VMEM((CHUNK, LBL), jnp.float32)] * 5,
            [pltpu.VMEM((CHUNK, LBL), jnp.float32)] * 3,
            pltpu.VMEM((STAGE, LBL), jnp.float32),
            pltpu.VMEM_SHARED((N_NODES, LBL), jnp.float32),
            [pltpu.SemaphoreType.DMA] * 5,
            [pltpu.SemaphoreType.DMA] * 3,
            [pltpu.SemaphoreType.DMA] * 2,
        ],
    )(pred, stg)


_MM_BLK = 2000


def _mm_body(h1_ref, w2_ref, out_ref):
    acc = jnp.zeros((_MM_BLK, LBL), jnp.float32)
    for fcc in range(4):
        acc = acc + jnp.maximum(h1_ref[fcc], 0.0) @ w2_ref[fcc]
    out_ref[...] = acc


def _relu_matmul(h1, w2r):
    return pl.pallas_call(
        _mm_body,
        grid=(N_NODES // _MM_BLK,),
        in_specs=[
            pl.BlockSpec((4, _MM_BLK, LBL), lambda i: (0, i, 0)),
            pl.BlockSpec((4, LBL, LBL), lambda i: (0, 0, 0)),
        ],
        out_specs=pl.BlockSpec((_MM_BLK, LBL), lambda i: (i, 0)),
        out_shape=jax.ShapeDtypeStruct((N_NODES, LBL), jnp.float32),
    )(h1, w2r)


_CB_ROWS = 12500


def _comb_body(p_ref, h2_ref, out_ref):
    out_ref[...] = (1.0 - ALPHA) * (p_ref[0] + p_ref[1]) + ALPHA * h2_ref[...]


def _combine(parts128, h2_128):
    return pl.pallas_call(
        _comb_body,
        out_shape=jax.ShapeDtypeStruct((_CB_ROWS, 128), jnp.float32),
    )(parts128, h2_128)


_SM_BLK = 2000


def _comb_sm_body(p_ref, h2_ref, out_ref):
    x = (1.0 - ALPHA) * (p_ref[0] + p_ref[1]) + ALPHA * h2_ref[...]
    m = jnp.max(x, axis=1, keepdims=True)
    e = jnp.exp(x - m)
    lse = jnp.log(jnp.sum(e, axis=1, keepdims=True)) + m
    out_ref[...] = x - lse


def _combine_softmax(parts, h2):
    return pl.pallas_call(
        _comb_sm_body,
        grid=(N_NODES // _SM_BLK,),
        in_specs=[
            pl.BlockSpec((NC, _SM_BLK, LBL), lambda i: (0, i, 0)),
            pl.BlockSpec((_SM_BLK, LBL), lambda i: (i, 0)),
        ],
        out_specs=pl.BlockSpec((_SM_BLK, LBL), lambda i: (i, 0)),
        out_shape=jax.ShapeDtypeStruct((N_NODES, LBL), jnp.float32),
    )(parts, h2)


def _pad_to(x, mult, fill_idx=False):
    n = x.shape[0]
    target = -(-n // mult) * mult
    if target == n:
        return x
    pad = target - n
    if fill_idx:
        extra = (jnp.arange(pad, dtype=jnp.int32) * 997) % N_NODES
    else:
        extra = jnp.zeros((pad,), x.dtype)
    return jnp.concatenate([x, extra])


def kernel(features_indices, feature_values, edge_indices, edge_weights, W1, W2):
    rid = features_indices[0].astype(jnp.int32)
    cid = features_indices[1].astype(jnp.int32)
    val = feature_values
    dst = edge_indices[0].astype(jnp.int32)
    src = edge_indices[1].astype(jnp.int32)

    fm = NS * CHUNK * FEAT_BATCH * 2
    cid2d = _pad_to(cid, fm).reshape(-1, CHUNK)
    rid2d = _pad_to(rid, fm, fill_idx=True).reshape(-1, CHUNK)
    val2d = jax.lax.bitcast_convert_type(
        _pad_to(val, fm).reshape(-1, CHUNK), jnp.int32)
    fstg = jnp.stack([cid2d, rid2d, val2d], axis=1)

    em = NW * CHUNK * EDGE_BATCH * 2
    src2d = _pad_to(src, em, fill_idx=True).reshape(-1, CHUNK)
    dst2d = _pad_to(dst, em, fill_idx=True).reshape(-1, CHUNK)
    w2d = jax.lax.bitcast_convert_type(
        _pad_to(edge_weights, em).reshape(-1, CHUNK), jnp.int32)
    estg = jnp.stack([src2d, dst2d, w2d], axis=1)

    w1r = W1.reshape(F_IN, 4, LBL).transpose(1, 0, 2)
    w2r = W2.reshape(4, LBL, LBL)

    h1 = _feat_spmm(w1r, fstg)
    h2 = _relu_matmul(h1, w2r)

    h2_128 = h2.reshape(_CB_ROWS, 128)
    pred = h2
    for it in range(N_ITERS):
        parts = _edge_spmm(pred, estg)
        if it + 1 < N_ITERS:
            pred = _combine(parts.reshape(NC, _CB_ROWS, 128), h2_128)
            pred = pred.reshape(N_NODES, LBL)
        else:
            pred = _combine_softmax(parts, h2)
    return pred


# speedup vs baseline: 1.0330x; 1.0330x over previous
import functools

import jax
import jax.numpy as jnp
from jax import lax
from jax.experimental import pallas as pl
from jax.experimental.pallas import tpu as pltpu
from jax.experimental.pallas import tpu_sc as plsc

N_NODES = 100000
F_IN = 128
HID = 64
LBL = 16
ALPHA = 0.1
N_ITERS = 10

NC = 2
NS = 16
NW = NC * NS

CHUNK = 128
ROWS_PER_TILE = N_NODES // NS
STAGE = 250

EDGE_BATCH = 16
FEAT_BATCH = 16

_mesh = plsc.VectorSubcoreMesh(core_axis_name="c", subcore_axis_name="s")


def _zero_own_slice(stage_v, acc_sh, s):
    hop = stage_v.shape[0]

    def zrow(i, _):
        stage_v[i] = jnp.zeros((LBL,), jnp.float32)
        return 0
    lax.fori_loop(0, hop, zrow, 0)

    def zcopy(k, _):
        pltpu.sync_copy(stage_v, acc_sh.at[pl.ds(s * ROWS_PER_TILE + k * hop, hop)])
        return 0
    lax.fori_loop(0, ROWS_PER_TILE // hop, zcopy, 0)


def _dump_own_slice(stage_v, acc_sh, out2d, s):
    hop = stage_v.shape[0]

    def dcopy(k, _):
        off = s * ROWS_PER_TILE + k * hop
        pltpu.sync_copy(acc_sh.at[pl.ds(off, hop)], stage_v)
        pltpu.sync_copy(stage_v, out2d.at[pl.ds(off, hop)])
        return 0
    lax.fori_loop(0, ROWS_PER_TILE // hop, dcopy, 0)


def _pipelined_chunks(nchunks, operand, sb, gbufs, obufs, acc_sh, gsem, ssem):
    ng, no = len(gbufs), len(obufs)
    gd, sd = {}, {}
    for h in range(ng - 1):
        gd[h] = pltpu.async_copy(operand.at[sb.at[h, 0]], gbufs[h], gsem[h])
    for j in range(nchunks):
        if j >= no:
            sd[j - no].wait()
        nxt = j + ng - 1
        if nxt < nchunks:
            gd[nxt] = pltpu.async_copy(
                operand.at[sb.at[nxt, 0]], gbufs[nxt % ng], gsem[nxt % ng])
        gd[j].wait()
        gb = gbufs[j % ng]
        ob = obufs[j % no]

        def scale(g, _, gb=gb, ob=ob, j=j):
            base16 = g * 16
            wv = plsc.bitcast(sb[j, 2, pl.ds(base16, 16)], jnp.float32)
            for i in range(16):
                ob[base16 + i] = gb[base16 + i] * wv[i]
            return 0
        lax.fori_loop(0, CHUNK // 16, scale, 0)
        sd[j] = pltpu.async_copy(
            ob, acc_sh.at[sb.at[j, 1]], ssem[j % no], add=True)
    for t in range(no):
        sd[nchunks - no + t].wait()


def _batches_with_prefetch(n_batches, nchunks, stg_hbm, first_chunk_fn,
                           operand, sbufs, gbufs, obufs, acc_sh,
                           gsem, ssem, isem):
    pltpu.async_copy(
        stg_hbm.at[pl.ds(first_chunk_fn(0), nchunks)], sbufs[0], isem[0])

    def pair_body(b2, _):
        for q in range(2):
            b = b2 * 2 + q
            nb = b + 1

            @pl.when(nb < n_batches)
            def _():
                pltpu.async_copy(
                    stg_hbm.at[pl.ds(first_chunk_fn(nb), nchunks)],
                    sbufs[1 - q], isem[1 - q])
            pltpu.make_async_copy(
                stg_hbm.at[pl.ds(0, nchunks)], sbufs[q], isem[q]).wait()
            _pipelined_chunks(nchunks, operand, sbufs[q], gbufs, obufs,
                              acc_sh, gsem, ssem)
        return 0
    lax.fori_loop(0, n_batches // 2, pair_body, 0)


def _feat_body(nnz_chunks_per_tile, w1r_hbm, stg_hbm, h1_hbm,
               w1_v, sbufs, gbufs, obufs, stage_v,
               w1_sh, acc_sh, gsem, ssem, isem):
    c = lax.axis_index("c")
    s = lax.axis_index("s")
    n_batches = nnz_chunks_per_tile // FEAT_BATCH

    for p in range(2):
        fc = c * 2 + p
        _zero_own_slice(stage_v, acc_sh, s)

        @pl.when(s == 0)
        def _():
            pltpu.sync_copy(w1r_hbm.at[fc], w1_v)
            pltpu.sync_copy(w1_v, w1_sh)
        plsc.subcore_barrier()

        def first_chunk(b):
            return s * nnz_chunks_per_tile + b * FEAT_BATCH
        _batches_with_prefetch(n_batches, FEAT_BATCH, stg_hbm, first_chunk,
                               w1_sh, sbufs, gbufs, obufs, acc_sh,
                               gsem, ssem, isem)
        plsc.subcore_barrier()

        _dump_own_slice(stage_v, acc_sh, h1_hbm.at[fc], s)


def _feat_spmm(w1r, stg):
    nnz_chunks = stg.shape[0]
    per_tile = nnz_chunks // NS
    body = functools.partial(_feat_body, per_tile)
    return pl.kernel(
        body,
        out_type=jax.ShapeDtypeStruct((4, N_NODES, LBL), jnp.float32),
        mesh=_mesh,
        compiler_params=pltpu.CompilerParams(use_tc_tiling_on_sc=False, needs_layout_passes=False),
        scratch_types=[
            pltpu.VMEM((F_IN, LBL), jnp.float32),
            [pltpu.VMEM((FEAT_BATCH, 3, CHUNK), jnp.int32)] * 2,
            [pltpu.VMEM((CHUNK, LBL), jnp.float32)] * 4,
            [pltpu.VMEM((CHUNK, LBL), jnp.float32)] * 3,
            pltpu.VMEM((STAGE // 2, LBL), jnp.float32),
            pltpu.VMEM_SHARED((F_IN, LBL), jnp.float32),
            pltpu.VMEM_SHARED((N_NODES, LBL), jnp.float32),
            [pltpu.SemaphoreType.DMA] * 4,
            [pltpu.SemaphoreType.DMA] * 3,
            [pltpu.SemaphoreType.DMA] * 2,
        ],
    )(w1r, stg)


def _edge_body(chunks_per_tile, pred_hbm, stg_hbm, part_hbm,
               sbufs, gbufs, obufs, stage_v, acc_sh, gsem, ssem, isem):
    c = lax.axis_index("c")
    s = lax.axis_index("s")
    wid = s * NC + c
    n_batches = chunks_per_tile // EDGE_BATCH

    _zero_own_slice(stage_v, acc_sh, s)
    plsc.subcore_barrier()

    def first_chunk(b):
        return wid * chunks_per_tile + b * EDGE_BATCH
    _batches_with_prefetch(n_batches, EDGE_BATCH, stg_hbm, first_chunk,
                           pred_hbm, sbufs, gbufs, obufs, acc_sh,
                           gsem, ssem, isem)
    plsc.subcore_barrier()

    _dump_own_slice(stage_v, acc_sh, part_hbm.at[c], s)


def _edge_spmm(pred, stg):
    chunks = stg.shape[0]
    per_tile = chunks // NW
    body = functools.partial(_edge_body, per_tile)
    return pl.kernel(
        body,
        out_type=jax.ShapeDtypeStruct((NC, N_NODES, LBL), jnp.float32),
        mesh=_mesh,
        compiler_params=pltpu.CompilerParams(use_tc_tiling_on_sc=False, needs_layout_passes=False),
        scratch_types=[
            [pltpu.VMEM((EDGE_BATCH, 3, CHUNK), jnp.int32)] * 2,
            [pltpu.VMEM((CHUNK, LBL), jnp.float32)] * 4,
            [pltpu.VMEM((CHUNK, LBL), jnp.float32)] * 3,
            pltpu.VMEM((STAGE, LBL), jnp.float32),
            pltpu.VMEM_SHARED((N_NODES, LBL), jnp.float32),
            [pltpu.SemaphoreType.DMA] * 4,
            [pltpu.SemaphoreType.DMA] * 3,
            [pltpu.SemaphoreType.DMA] * 2,
        ],
    )(pred, stg)


_MM_BLK = 2000


def _mm_body(h1_ref, w2_ref, out_ref):
    acc = jnp.zeros((_MM_BLK, LBL), jnp.float32)
    for fcc in range(4):
        acc = acc + jnp.maximum(h1_ref[fcc], 0.0) @ w2_ref[fcc]
    out_ref[...] = acc


def _relu_matmul(h1, w2r):
    return pl.pallas_call(
        _mm_body,
        grid=(N_NODES // _MM_BLK,),
        in_specs=[
            pl.BlockSpec((4, _MM_BLK, LBL), lambda i: (0, i, 0)),
            pl.BlockSpec((4, LBL, LBL), lambda i: (0, 0, 0)),
        ],
        out_specs=pl.BlockSpec((_MM_BLK, LBL), lambda i: (i, 0)),
        out_shape=jax.ShapeDtypeStruct((N_NODES, LBL), jnp.float32),
    )(h1, w2r)


_CB_ROWS = 12500


def _comb_body(p_ref, h2_ref, out_ref):
    out_ref[...] = (1.0 - ALPHA) * (p_ref[0] + p_ref[1]) + ALPHA * h2_ref[...]


def _combine(parts128, h2_128):
    return pl.pallas_call(
        _comb_body,
        out_shape=jax.ShapeDtypeStruct((_CB_ROWS, 128), jnp.float32),
    )(parts128, h2_128)


_SM_BLK = 2000


def _comb_sm_body(p_ref, h2_ref, out_ref):
    x = (1.0 - ALPHA) * (p_ref[0] + p_ref[1]) + ALPHA * h2_ref[...]
    m = jnp.max(x, axis=1, keepdims=True)
    e = jnp.exp(x - m)
    lse = jnp.log(jnp.sum(e, axis=1, keepdims=True)) + m
    out_ref[...] = x - lse


def _combine_softmax(parts, h2):
    return pl.pallas_call(
        _comb_sm_body,
        grid=(N_NODES // _SM_BLK,),
        in_specs=[
            pl.BlockSpec((NC, _SM_BLK, LBL), lambda i: (0, i, 0)),
            pl.BlockSpec((_SM_BLK, LBL), lambda i: (i, 0)),
        ],
        out_specs=pl.BlockSpec((_SM_BLK, LBL), lambda i: (i, 0)),
        out_shape=jax.ShapeDtypeStruct((N_NODES, LBL), jnp.float32),
    )(parts, h2)


def _pad_to(x, mult, fill_idx=False):
    n = x.shape[0]
    target = -(-n // mult) * mult
    if target == n:
        return x
    pad = target - n
    if fill_idx:
        extra = (jnp.arange(pad, dtype=jnp.int32) * 997) % N_NODES
    else:
        extra = jnp.zeros((pad,), x.dtype)
    return jnp.concatenate([x, extra])


def kernel(features_indices, feature_values, edge_indices, edge_weights, W1, W2):
    rid = features_indices[0].astype(jnp.int32)
    cid = features_indices[1].astype(jnp.int32)
    val = feature_values
    dst = edge_indices[0].astype(jnp.int32)
    src = edge_indices[1].astype(jnp.int32)

    fm = NS * CHUNK * FEAT_BATCH * 2
    cid2d = _pad_to(cid, fm).reshape(-1, CHUNK)
    rid2d = _pad_to(rid, fm, fill_idx=True).reshape(-1, CHUNK)
    val2d = jax.lax.bitcast_convert_type(
        _pad_to(val, fm).reshape(-1, CHUNK), jnp.int32)
    fstg = jnp.stack([cid2d, rid2d, val2d], axis=1)

    em = NW * CHUNK * EDGE_BATCH * 2
    src2d = _pad_to(src, em, fill_idx=True).reshape(-1, CHUNK)
    dst2d = _pad_to(dst, em, fill_idx=True).reshape(-1, CHUNK)
    w2d = jax.lax.bitcast_convert_type(
        _pad_to(edge_weights, em).reshape(-1, CHUNK), jnp.int32)
    estg = jnp.stack([src2d, dst2d, w2d], axis=1)

    w1r = W1.reshape(F_IN, 4, LBL).transpose(1, 0, 2)
    w2r = W2.reshape(4, LBL, LBL)

    h1 = _feat_spmm(w1r, fstg)
    h2 = _relu_matmul(h1, w2r)

    h2_128 = h2.reshape(_CB_ROWS, 128)
    pred = h2
    for it in range(N_ITERS):
        parts = _edge_spmm(pred, estg)
        if it + 1 < N_ITERS:
            pred = _combine(parts.reshape(NC, _CB_ROWS, 128), h2_128)
            pred = pred.reshape(N_NODES, LBL)
        else:
            pred = _combine_softmax(parts, h2)
    return pred


# speedup vs baseline: 1.0815x; 1.0469x over previous
import functools

import jax
import jax.numpy as jnp
from jax import lax
from jax.experimental import pallas as pl
from jax.experimental.pallas import tpu as pltpu
from jax.experimental.pallas import tpu_sc as plsc

N_NODES = 100000
F_IN = 128
HID = 64
LBL = 16
ALPHA = 0.1
N_ITERS = 10

NC = 2
NS = 16
NW = NC * NS

CHUNK = 128
ROWS_PER_TILE = N_NODES // NS
STAGE = 250

EDGE_BATCH = 16
FEAT_BATCH = 16

_mesh = plsc.VectorSubcoreMesh(core_axis_name="c", subcore_axis_name="s")


def _zero_own_slice(stage_v, acc_sh, s):
    hop = stage_v.shape[0]

    def zrow(i, _):
        stage_v[i] = jnp.zeros((LBL,), jnp.float32)
        return 0
    lax.fori_loop(0, hop, zrow, 0)

    def zcopy(k, _):
        pltpu.sync_copy(stage_v, acc_sh.at[pl.ds(s * ROWS_PER_TILE + k * hop, hop)])
        return 0
    lax.fori_loop(0, ROWS_PER_TILE // hop, zcopy, 0)


def _dump_own_slice(stage_v, acc_sh, out2d, s):
    hop = stage_v.shape[0]

    def dcopy(k, _):
        off = s * ROWS_PER_TILE + k * hop
        pltpu.sync_copy(acc_sh.at[pl.ds(off, hop)], stage_v)
        pltpu.sync_copy(stage_v, out2d.at[pl.ds(off, hop)])
        return 0
    lax.fori_loop(0, ROWS_PER_TILE // hop, dcopy, 0)


def _pipelined_chunks(nchunks, operand, sb, gbufs, obufs, acc_sh, gsem, ssem):
    ng, no = len(gbufs), len(obufs)
    gd, sd = {}, {}
    for h in range(ng - 1):
        gd[h] = pltpu.async_copy(operand.at[sb.at[h, 0]], gbufs[h], gsem[h])
    for j in range(nchunks):
        if j >= no:
            sd[j - no].wait()
        nxt = j + ng - 1
        if nxt < nchunks:
            gd[nxt] = pltpu.async_copy(
                operand.at[sb.at[nxt, 0]], gbufs[nxt % ng], gsem[nxt % ng])
        gd[j].wait()
        gb = gbufs[j % ng]
        ob = obufs[j % no]

        def scale(g, _, gb=gb, ob=ob, j=j):
            base16 = g * 16
            wv = plsc.bitcast(sb[j, 2, pl.ds(base16, 16)], jnp.float32)
            for i in range(16):
                ob[base16 + i] = gb[base16 + i] * wv[i]
            return 0
        lax.fori_loop(0, CHUNK // 16, scale, 0)
        sd[j] = pltpu.async_copy(
            ob, acc_sh.at[sb.at[j, 1]], ssem[j % no], add=True)
    for t in range(no):
        sd[nchunks - no + t].wait()


def _batches_with_prefetch(n_batches, nchunks, stg_hbm, first_chunk_fn,
                           operand, sbufs, gbufs, obufs, acc_sh,
                           gsem, ssem, isem):
    npairs = n_batches // 2
    ng, no = len(gbufs), len(obufs)
    nch2 = nchunks * 2
    pltpu.async_copy(
        stg_hbm.at[pl.ds(first_chunk_fn(0), nchunks)], sbufs[0], isem[0])

    def pair_body(p, _):
        pltpu.async_copy(
            stg_hbm.at[pl.ds(first_chunk_fn(2 * p + 1), nchunks)],
            sbufs[1], isem[1])
        pltpu.make_async_copy(
            stg_hbm.at[pl.ds(0, nchunks)], sbufs[0], isem[0]).wait()

        def sbref(j, plane):
            return sbufs[j // nchunks].at[j % nchunks, plane]

        gd, sd = {}, {}
        for h in range(ng - 1):
            gd[h] = pltpu.async_copy(operand.at[sbref(h, 0)], gbufs[h], gsem[h])
        for j in range(nch2):
            if j == nchunks - (ng - 1):
                pltpu.make_async_copy(
                    stg_hbm.at[pl.ds(0, nchunks)], sbufs[1], isem[1]).wait()
            if j == nchunks + no:
                @pl.when(p + 1 < npairs)
                def _():
                    pltpu.async_copy(
                        stg_hbm.at[pl.ds(first_chunk_fn(2 * p + 2), nchunks)],
                        sbufs[0], isem[0])
            if j >= no:
                sd[j - no].wait()
            nxt = j + ng - 1
            if nxt < nch2:
                gd[nxt] = pltpu.async_copy(
                    operand.at[sbref(nxt, 0)], gbufs[nxt % ng], gsem[nxt % ng])
            gd[j].wait()
            gb = gbufs[j % ng]
            ob = obufs[j % no]

            def scale(g, _, gb=gb, ob=ob, j=j):
                base16 = g * 16
                wv = plsc.bitcast(
                    sbufs[j // nchunks][j % nchunks, 2, pl.ds(base16, 16)],
                    jnp.float32)
                for i in range(16):
                    ob[base16 + i] = gb[base16 + i] * wv[i]
                return 0
            lax.fori_loop(0, CHUNK // 16, scale, 0)
            sd[j] = pltpu.async_copy(
                ob, acc_sh.at[sbref(j, 1)], ssem[j % no], add=True)
        for t in range(no):
            sd[nch2 - no + t].wait()
        return 0
    lax.fori_loop(0, npairs, pair_body, 0)


def _feat_body(nnz_chunks_per_tile, w1r_hbm, stg_hbm, h1_hbm,
               w1_v, sbufs, gbufs, obufs, stage_v,
               w1_sh, acc_sh, gsem, ssem, isem):
    c = lax.axis_index("c")
    s = lax.axis_index("s")
    n_batches = nnz_chunks_per_tile // FEAT_BATCH

    for p in range(2):
        fc = c * 2 + p
        _zero_own_slice(stage_v, acc_sh, s)

        @pl.when(s == 0)
        def _():
            pltpu.sync_copy(w1r_hbm.at[fc], w1_v)
            pltpu.sync_copy(w1_v, w1_sh)
        plsc.subcore_barrier()

        def first_chunk(b):
            return s * nnz_chunks_per_tile + b * FEAT_BATCH
        _batches_with_prefetch(n_batches, FEAT_BATCH, stg_hbm, first_chunk,
                               w1_sh, sbufs, gbufs, obufs, acc_sh,
                               gsem, ssem, isem)
        plsc.subcore_barrier()

        _dump_own_slice(stage_v, acc_sh, h1_hbm.at[fc], s)


def _feat_spmm(w1r, stg):
    nnz_chunks = stg.shape[0]
    per_tile = nnz_chunks // NS
    body = functools.partial(_feat_body, per_tile)
    return pl.kernel(
        body,
        out_type=jax.ShapeDtypeStruct((4, N_NODES, LBL), jnp.float32),
        mesh=_mesh,
        compiler_params=pltpu.CompilerParams(use_tc_tiling_on_sc=False, needs_layout_passes=False),
        scratch_types=[
            pltpu.VMEM((F_IN, LBL), jnp.float32),
            [pltpu.VMEM((FEAT_BATCH, 3, CHUNK), jnp.int32)] * 2,
            [pltpu.VMEM((CHUNK, LBL), jnp.float32)] * 4,
            [pltpu.VMEM((CHUNK, LBL), jnp.float32)] * 3,
            pltpu.VMEM((STAGE // 2, LBL), jnp.float32),
            pltpu.VMEM_SHARED((F_IN, LBL), jnp.float32),
            pltpu.VMEM_SHARED((N_NODES, LBL), jnp.float32),
            [pltpu.SemaphoreType.DMA] * 4,
            [pltpu.SemaphoreType.DMA] * 3,
            [pltpu.SemaphoreType.DMA] * 2,
        ],
    )(w1r, stg)


def _edge_body(chunks_per_tile, pred_hbm, stg_hbm, part_hbm,
               sbufs, gbufs, obufs, stage_v, acc_sh, gsem, ssem, isem):
    c = lax.axis_index("c")
    s = lax.axis_index("s")
    wid = s * NC + c
    n_batches = chunks_per_tile // EDGE_BATCH

    _zero_own_slice(stage_v, acc_sh, s)
    plsc.subcore_barrier()

    def first_chunk(b):
        return wid * chunks_per_tile + b * EDGE_BATCH
    _batches_with_prefetch(n_batches, EDGE_BATCH, stg_hbm, first_chunk,
                           pred_hbm, sbufs, gbufs, obufs, acc_sh,
                           gsem, ssem, isem)
    plsc.subcore_barrier()

    _dump_own_slice(stage_v, acc_sh, part_hbm.at[c], s)


def _edge_spmm(pred, stg):
    chunks = stg.shape[0]
    per_tile = chunks // NW
    body = functools.partial(_edge_body, per_tile)
    return pl.kernel(
        body,
        out_type=jax.ShapeDtypeStruct((NC, N_NODES, LBL), jnp.float32),
        mesh=_mesh,
        compiler_params=pltpu.CompilerParams(use_tc_tiling_on_sc=False, needs_layout_passes=False),
        scratch_types=[
            [pltpu.VMEM((EDGE_BATCH, 3, CHUNK), jnp.int32)] * 2,
            [pltpu.VMEM((CHUNK, LBL), jnp.float32)] * 4,
            [pltpu.VMEM((CHUNK, LBL), jnp.float32)] * 3,
            pltpu.VMEM((STAGE, LBL), jnp.float32),
            pltpu.VMEM_SHARED((N_NODES, LBL), jnp.float32),
            [pltpu.SemaphoreType.DMA] * 4,
            [pltpu.SemaphoreType.DMA] * 3,
            [pltpu.SemaphoreType.DMA] * 2,
        ],
    )(pred, stg)


_MM_BLK = 2000


def _mm_body(h1_ref, w2_ref, out_ref):
    acc = jnp.zeros((_MM_BLK, LBL), jnp.float32)
    for fcc in range(4):
        acc = acc + jnp.maximum(h1_ref[fcc], 0.0) @ w2_ref[fcc]
    out_ref[...] = acc


def _relu_matmul(h1, w2r):
    return pl.pallas_call(
        _mm_body,
        grid=(N_NODES // _MM_BLK,),
        in_specs=[
            pl.BlockSpec((4, _MM_BLK, LBL), lambda i: (0, i, 0)),
            pl.BlockSpec((4, LBL, LBL), lambda i: (0, 0, 0)),
        ],
        out_specs=pl.BlockSpec((_MM_BLK, LBL), lambda i: (i, 0)),
        out_shape=jax.ShapeDtypeStruct((N_NODES, LBL), jnp.float32),
    )(h1, w2r)


_CB_ROWS = 12500


def _comb_body(p_ref, h2_ref, out_ref):
    out_ref[...] = (1.0 - ALPHA) * (p_ref[0] + p_ref[1]) + ALPHA * h2_ref[...]


def _combine(parts128, h2_128):
    return pl.pallas_call(
        _comb_body,
        out_shape=jax.ShapeDtypeStruct((_CB_ROWS, 128), jnp.float32),
    )(parts128, h2_128)


_SM_BLK = 2000


def _comb_sm_body(p_ref, h2_ref, out_ref):
    x = (1.0 - ALPHA) * (p_ref[0] + p_ref[1]) + ALPHA * h2_ref[...]
    m = jnp.max(x, axis=1, keepdims=True)
    e = jnp.exp(x - m)
    lse = jnp.log(jnp.sum(e, axis=1, keepdims=True)) + m
    out_ref[...] = x - lse


def _combine_softmax(parts, h2):
    return pl.pallas_call(
        _comb_sm_body,
        grid=(N_NODES // _SM_BLK,),
        in_specs=[
            pl.BlockSpec((NC, _SM_BLK, LBL), lambda i: (0, i, 0)),
            pl.BlockSpec((_SM_BLK, LBL), lambda i: (i, 0)),
        ],
        out_specs=pl.BlockSpec((_SM_BLK, LBL), lambda i: (i, 0)),
        out_shape=jax.ShapeDtypeStruct((N_NODES, LBL), jnp.float32),
    )(parts, h2)


def _pad_to(x, mult, fill_idx=False):
    n = x.shape[0]
    target = -(-n // mult) * mult
    if target == n:
        return x
    pad = target - n
    if fill_idx:
        extra = (jnp.arange(pad, dtype=jnp.int32) * 997) % N_NODES
    else:
        extra = jnp.zeros((pad,), x.dtype)
    return jnp.concatenate([x, extra])


def kernel(features_indices, feature_values, edge_indices, edge_weights, W1, W2):
    rid = features_indices[0].astype(jnp.int32)
    cid = features_indices[1].astype(jnp.int32)
    val = feature_values
    dst = edge_indices[0].astype(jnp.int32)
    src = edge_indices[1].astype(jnp.int32)

    fm = NS * CHUNK * FEAT_BATCH * 2
    cid2d = _pad_to(cid, fm).reshape(-1, CHUNK)
    rid2d = _pad_to(rid, fm, fill_idx=True).reshape(-1, CHUNK)
    val2d = jax.lax.bitcast_convert_type(
        _pad_to(val, fm).reshape(-1, CHUNK), jnp.int32)
    fstg = jnp.stack([cid2d, rid2d, val2d], axis=1)

    em = NW * CHUNK * EDGE_BATCH * 2
    src2d = _pad_to(src, em, fill_idx=True).reshape(-1, CHUNK)
    dst2d = _pad_to(dst, em, fill_idx=True).reshape(-1, CHUNK)
    w2d = jax.lax.bitcast_convert_type(
        _pad_to(edge_weights, em).reshape(-1, CHUNK), jnp.int32)
    estg = jnp.stack([src2d, dst2d, w2d], axis=1)

    w1r = W1.reshape(F_IN, 4, LBL).transpose(1, 0, 2)
    w2r = W2.reshape(4, LBL, LBL)

    h1 = _feat_spmm(w1r, fstg)
    h2 = _relu_matmul(h1, w2r)

    h2_128 = h2.reshape(_CB_ROWS, 128)
    pred = h2
    for it in range(N_ITERS):
        parts = _edge_spmm(pred, estg)
        if it + 1 < N_ITERS:
            pred = _combine(parts.reshape(NC, _CB_ROWS, 128), h2_128)
            pred = pred.reshape(N_NODES, LBL)
        else:
            pred = _combine_softmax(parts, h2)
    return pred


# speedup vs baseline: 1.1632x; 1.0755x over previous
import functools

import jax
import jax.numpy as jnp
from jax import lax
from jax.experimental import pallas as pl
from jax.experimental.pallas import tpu as pltpu
from jax.experimental.pallas import tpu_sc as plsc

N_NODES = 100000
F_IN = 128
HID = 64
LBL = 16
ALPHA = 0.1
N_ITERS = 10

NC = 2
NS = 16
NW = NC * NS

CHUNK = 128
ROWS_PER_TILE = N_NODES // NS
STAGE = 250

EDGE_BATCH = 16
FEAT_BATCH = 16

_mesh = plsc.VectorSubcoreMesh(core_axis_name="c", subcore_axis_name="s")


def _zero_own_slice(stage_v, acc_sh, s):
    hop = stage_v.shape[0]

    def zrow(i, _):
        stage_v[i] = jnp.zeros((LBL,), jnp.float32)
        return 0
    lax.fori_loop(0, hop, zrow, 0)

    def zcopy(k, _):
        pltpu.sync_copy(stage_v, acc_sh.at[pl.ds(s * ROWS_PER_TILE + k * hop, hop)])
        return 0
    lax.fori_loop(0, ROWS_PER_TILE // hop, zcopy, 0)


def _dump_own_slice(stage_v, acc_sh, out2d, s):
    hop = stage_v.shape[0]

    def dcopy(k, _):
        off = s * ROWS_PER_TILE + k * hop
        pltpu.sync_copy(acc_sh.at[pl.ds(off, hop)], stage_v)
        pltpu.sync_copy(stage_v, out2d.at[pl.ds(off, hop)])
        return 0
    lax.fori_loop(0, ROWS_PER_TILE // hop, dcopy, 0)


def _pipelined_chunks(nchunks, operand, sb, gbufs, obufs, acc_sh, gsem, ssem):
    ng, no = len(gbufs), len(obufs)
    gd, sd = {}, {}
    for h in range(ng - 1):
        gd[h] = pltpu.async_copy(operand.at[sb.at[h, 0]], gbufs[h], gsem[h])
    for j in range(nchunks):
        if j >= no:
            sd[j - no].wait()
        nxt = j + ng - 1
        if nxt < nchunks:
            gd[nxt] = pltpu.async_copy(
                operand.at[sb.at[nxt, 0]], gbufs[nxt % ng], gsem[nxt % ng])
        gd[j].wait()
        gb = gbufs[j % ng]
        ob = obufs[j % no]

        def scale(g, _, gb=gb, ob=ob, j=j):
            base16 = g * 16
            wv = plsc.bitcast(sb[j, 2, pl.ds(base16, 16)], jnp.float32)
            for i in range(16):
                ob[base16 + i] = gb[base16 + i] * wv[i]
            return 0
        lax.fori_loop(0, CHUNK // 16, scale, 0)
        sd[j] = pltpu.async_copy(
            ob, acc_sh.at[sb.at[j, 1]], ssem[j % no], add=True)
    for t in range(no):
        sd[nchunks - no + t].wait()


def _batches_with_prefetch(n_batches, nchunks, stg_hbm, first_chunk_fn,
                           operand, sbufs, gbufs, obufs, acc_sh,
                           gsem, ssem, isem):
    npairs = n_batches // 2
    ng, no = len(gbufs), len(obufs)
    nch2 = nchunks * 2
    pltpu.async_copy(
        stg_hbm.at[pl.ds(first_chunk_fn(0), nchunks)], sbufs[0], isem[0])

    def pair_body(p, _):
        pltpu.async_copy(
            stg_hbm.at[pl.ds(first_chunk_fn(2 * p + 1), nchunks)],
            sbufs[1], isem[1])
        pltpu.make_async_copy(
            stg_hbm.at[pl.ds(0, nchunks)], sbufs[0], isem[0]).wait()

        def sbref(j, plane):
            return sbufs[j // nchunks].at[j % nchunks, plane]

        gd, sd = {}, {}
        for h in range(ng - 1):
            gd[h] = pltpu.async_copy(operand.at[sbref(h, 0)], gbufs[h], gsem[h])
        for j in range(nch2):
            if j == nchunks - (ng - 1):
                pltpu.make_async_copy(
                    stg_hbm.at[pl.ds(0, nchunks)], sbufs[1], isem[1]).wait()
            if j == nchunks + no:
                @pl.when(p + 1 < npairs)
                def _():
                    pltpu.async_copy(
                        stg_hbm.at[pl.ds(first_chunk_fn(2 * p + 2), nchunks)],
                        sbufs[0], isem[0])
            if j >= no:
                sd[j - no].wait()
            nxt = j + ng - 1
            if nxt < nch2:
                gd[nxt] = pltpu.async_copy(
                    operand.at[sbref(nxt, 0)], gbufs[nxt % ng], gsem[nxt % ng])
            gd[j].wait()
            gb = gbufs[j % ng]
            ob = obufs[j % no]

            def scale(g, _, gb=gb, ob=ob, j=j):
                base16 = g * 16
                wv = plsc.bitcast(
                    sbufs[j // nchunks][j % nchunks, 2, pl.ds(base16, 16)],
                    jnp.float32)
                for i in range(16):
                    ob[base16 + i] = gb[base16 + i] * wv[i]
                return 0
            lax.fori_loop(0, CHUNK // 16, scale, 0)
            sd[j] = pltpu.async_copy(
                ob, acc_sh.at[sbref(j, 1)], ssem[j % no], add=True)
        for t in range(no):
            sd[nch2 - no + t].wait()
        return 0
    lax.fori_loop(0, npairs, pair_body, 0)


def _feat_body(nnz_chunks_per_tile, w1r_hbm, stg_hbm, h1_hbm,
               w1_v, sbufs, gbufs, obufs, stage_v,
               w1_sh, acc_sh, gsem, ssem, isem):
    c = lax.axis_index("c")
    s = lax.axis_index("s")
    n_batches = nnz_chunks_per_tile // FEAT_BATCH

    for p in range(2):
        fc = c * 2 + p
        _zero_own_slice(stage_v, acc_sh, s)

        @pl.when(s == 0)
        def _():
            pltpu.sync_copy(w1r_hbm.at[fc], w1_v)
            pltpu.sync_copy(w1_v, w1_sh)
        plsc.subcore_barrier()

        def first_chunk(b):
            return s * nnz_chunks_per_tile + b * FEAT_BATCH
        _batches_with_prefetch(n_batches, FEAT_BATCH, stg_hbm, first_chunk,
                               w1_sh, sbufs, gbufs, obufs, acc_sh,
                               gsem, ssem, isem)
        plsc.subcore_barrier()

        _dump_own_slice(stage_v, acc_sh, h1_hbm.at[fc], s)


def _feat_spmm(w1r, stg):
    nnz_chunks = stg.shape[0]
    per_tile = nnz_chunks // NS
    body = functools.partial(_feat_body, per_tile)
    return pl.kernel(
        body,
        out_type=jax.ShapeDtypeStruct((4, N_NODES, LBL), jnp.float32),
        mesh=_mesh,
        compiler_params=pltpu.CompilerParams(use_tc_tiling_on_sc=False, needs_layout_passes=False),
        scratch_types=[
            pltpu.VMEM((F_IN, LBL), jnp.float32),
            [pltpu.VMEM((FEAT_BATCH, 3, CHUNK), jnp.int32)] * 2,
            [pltpu.VMEM((CHUNK, LBL), jnp.float32)] * 4,
            [pltpu.VMEM((CHUNK, LBL), jnp.float32)] * 3,
            pltpu.VMEM((STAGE // 2, LBL), jnp.float32),
            pltpu.VMEM_SHARED((F_IN, LBL), jnp.float32),
            pltpu.VMEM_SHARED((N_NODES, LBL), jnp.float32),
            [pltpu.SemaphoreType.DMA] * 4,
            [pltpu.SemaphoreType.DMA] * 3,
            [pltpu.SemaphoreType.DMA] * 2,
        ],
    )(w1r, stg)


def _edge_body(chunks_per_tile, pred_hbm, stg_hbm, part_hbm,
               sbufs, gbufs, obufs, stage_v, acc_sh, gsem, ssem, isem):
    c = lax.axis_index("c")
    s = lax.axis_index("s")
    wid = s * NC + c
    n_batches = chunks_per_tile // EDGE_BATCH

    _zero_own_slice(stage_v, acc_sh, s)
    plsc.subcore_barrier()

    def first_chunk(b):
        return wid * chunks_per_tile + b * EDGE_BATCH
    _batches_with_prefetch(n_batches, EDGE_BATCH, stg_hbm, first_chunk,
                           pred_hbm, sbufs, gbufs, obufs, acc_sh,
                           gsem, ssem, isem)
    plsc.subcore_barrier()

    _dump_own_slice(stage_v, acc_sh, part_hbm.at[c], s)


def _edge_spmm(pred, stg):
    chunks = stg.shape[0]
    per_tile = chunks // NW
    body = functools.partial(_edge_body, per_tile)
    return pl.kernel(
        body,
        out_type=jax.ShapeDtypeStruct((NC, N_NODES, LBL), jnp.float32),
        mesh=_mesh,
        compiler_params=pltpu.CompilerParams(use_tc_tiling_on_sc=False, needs_layout_passes=False),
        scratch_types=[
            [pltpu.VMEM((EDGE_BATCH, 3, CHUNK), jnp.int32)] * 2,
            [pltpu.VMEM((CHUNK, LBL), jnp.float32)] * 5,
            [pltpu.VMEM((CHUNK, LBL), jnp.float32)] * 3,
            pltpu.VMEM((STAGE // 2, LBL), jnp.float32),
            pltpu.VMEM_SHARED((N_NODES, LBL), jnp.float32),
            [pltpu.SemaphoreType.DMA] * 5,
            [pltpu.SemaphoreType.DMA] * 3,
            [pltpu.SemaphoreType.DMA] * 2,
        ],
    )(pred, stg)


_MM_BLK = 2000


def _mm_body(h1_ref, w2_ref, out_ref):
    acc = jnp.zeros((_MM_BLK, LBL), jnp.float32)
    for fcc in range(4):
        acc = acc + jnp.maximum(h1_ref[fcc], 0.0) @ w2_ref[fcc]
    out_ref[...] = acc


def _relu_matmul(h1, w2r):
    return pl.pallas_call(
        _mm_body,
        grid=(N_NODES // _MM_BLK,),
        in_specs=[
            pl.BlockSpec((4, _MM_BLK, LBL), lambda i: (0, i, 0)),
            pl.BlockSpec((4, LBL, LBL), lambda i: (0, 0, 0)),
        ],
        out_specs=pl.BlockSpec((_MM_BLK, LBL), lambda i: (i, 0)),
        out_shape=jax.ShapeDtypeStruct((N_NODES, LBL), jnp.float32),
    )(h1, w2r)


_CB_ROWS = 12500


def _comb_body(p_ref, h2_ref, out_ref):
    out_ref[...] = (1.0 - ALPHA) * (p_ref[0] + p_ref[1]) + ALPHA * h2_ref[...]


def _combine(parts128, h2_128):
    return pl.pallas_call(
        _comb_body,
        out_shape=jax.ShapeDtypeStruct((_CB_ROWS, 128), jnp.float32),
    )(parts128, h2_128)


_SM_BLK = 2000


def _comb_sm_body(p_ref, h2_ref, out_ref):
    x = (1.0 - ALPHA) * (p_ref[0] + p_ref[1]) + ALPHA * h2_ref[...]
    m = jnp.max(x, axis=1, keepdims=True)
    e = jnp.exp(x - m)
    lse = jnp.log(jnp.sum(e, axis=1, keepdims=True)) + m
    out_ref[...] = x - lse


def _combine_softmax(parts, h2):
    return pl.pallas_call(
        _comb_sm_body,
        grid=(N_NODES // _SM_BLK,),
        in_specs=[
            pl.BlockSpec((NC, _SM_BLK, LBL), lambda i: (0, i, 0)),
            pl.BlockSpec((_SM_BLK, LBL), lambda i: (i, 0)),
        ],
        out_specs=pl.BlockSpec((_SM_BLK, LBL), lambda i: (i, 0)),
        out_shape=jax.ShapeDtypeStruct((N_NODES, LBL), jnp.float32),
    )(parts, h2)


def _pad_to(x, mult, fill_idx=False):
    n = x.shape[0]
    target = -(-n // mult) * mult
    if target == n:
        return x
    pad = target - n
    if fill_idx:
        extra = (jnp.arange(pad, dtype=jnp.int32) * 997) % N_NODES
    else:
        extra = jnp.zeros((pad,), x.dtype)
    return jnp.concatenate([x, extra])


def kernel(features_indices, feature_values, edge_indices, edge_weights, W1, W2):
    rid = features_indices[0].astype(jnp.int32)
    cid = features_indices[1].astype(jnp.int32)
    val = feature_values
    dst = edge_indices[0].astype(jnp.int32)
    src = edge_indices[1].astype(jnp.int32)

    fm = NS * CHUNK * FEAT_BATCH * 2
    cid2d = _pad_to(cid, fm).reshape(-1, CHUNK)
    rid2d = _pad_to(rid, fm, fill_idx=True).reshape(-1, CHUNK)
    val2d = jax.lax.bitcast_convert_type(
        _pad_to(val, fm).reshape(-1, CHUNK), jnp.int32)
    fstg = jnp.stack([cid2d, rid2d, val2d], axis=1)

    em = NW * CHUNK * EDGE_BATCH * 2
    src2d = _pad_to(src, em, fill_idx=True).reshape(-1, CHUNK)
    dst2d = _pad_to(dst, em, fill_idx=True).reshape(-1, CHUNK)
    w2d = jax.lax.bitcast_convert_type(
        _pad_to(edge_weights, em).reshape(-1, CHUNK), jnp.int32)
    estg = jnp.stack([src2d, dst2d, w2d], axis=1)

    w1r = W1.reshape(F_IN, 4, LBL).transpose(1, 0, 2)
    w2r = W2.reshape(4, LBL, LBL)

    h1 = _feat_spmm(w1r, fstg)
    h2 = _relu_matmul(h1, w2r)

    h2_128 = h2.reshape(_CB_ROWS, 128)
    pred = h2
    for it in range(N_ITERS):
        parts = _edge_spmm(pred, estg)
        if it + 1 < N_ITERS:
            pred = _combine(parts.reshape(NC, _CB_ROWS, 128), h2_128)
            pred = pred.reshape(N_NODES, LBL)
        else:
            pred = _combine_softmax(parts, h2)
    return pred
